# baseline (speedup 1.0000x reference)
import functools

import jax
import jax.numpy as jnp
from jax.experimental import pallas as pl
from jax.experimental.pallas import tpu as pltpu


def _pair_kernel(ids_ref, tbl_ref, o_ref, *, vocab, tile_s):
    k = pl.program_id(0)
    tok = jax.lax.broadcasted_iota(jnp.int32, (1, vocab), 1)
    ids_a = ids_ref[2 * k, :]
    ids_b = ids_ref[2 * k + 1, :]
    onehot = jnp.concatenate(
        [(ids_a[:, None] == tok).astype(jnp.bfloat16),
         (ids_b[:, None] == tok).astype(jnp.bfloat16)], axis=1)
    part = jnp.dot(onehot, tbl_ref[...].astype(jnp.bfloat16),
                   preferred_element_type=jnp.float32)

    @pl.when(k == 0)
    def _():
        o_ref[...] = part

    @pl.when(k > 0)
    def _():
        o_ref[...] += part


@functools.partial(jax.jit, static_argnames=("vocab",))
def _embed_sum(idx, tbl, *, vocab):
    n_rows, seq = idx.shape
    _, d = tbl.shape

    body = functools.partial(_pair_kernel, vocab=vocab, tile_s=seq)
    return pl.pallas_call(
        body,
        out_shape=jax.ShapeDtypeStruct((seq, d), jnp.float32),
        grid=(n_rows // 2,),
        in_specs=[
            pl.BlockSpec((n_rows, seq), lambda k: (0, 0)),
            pl.BlockSpec((2 * vocab, d), lambda k: (k, 0)),
        ],
        out_specs=pl.BlockSpec((seq, d), lambda k: (0, 0)),
        compiler_params=pltpu.CompilerParams(
            dimension_semantics=("arbitrary",),
            vmem_limit_bytes=64 * 2**20),
    )(idx, tbl)


def kernel(xi, tables):
    xi = jnp.asarray(xi)
    n_levels = xi.shape[-1] - 1
    idx = jnp.transpose(xi[:, :n_levels]).astype(jnp.int32)
    idx = jnp.concatenate(
        [idx, jnp.full((1, idx.shape[1]), -1, jnp.int32)])
    n_tbl, n_tok, d = tables.shape
    tbl = tables.reshape(n_tbl * n_tok, d)
    return _embed_sum(idx, tbl, vocab=n_tok)


# speedup vs baseline: 1.0686x; 1.0686x over previous
import functools

import jax
import jax.numpy as jnp
from jax.experimental import pallas as pl
from jax.experimental.pallas import tpu as pltpu


def _two_phase_kernel(ids_ref, tbl_ref, o_ref, cast_ref, *,
                      n_levels, n_tiles, vocab, tile_s):
    k = pl.program_id(0)

    @pl.when(k < n_levels)
    def _cast_level():
        off = pl.multiple_of(k * vocab, vocab)
        cast_ref[pl.ds(off, vocab), :] = tbl_ref[...].astype(jnp.bfloat16)

    for m in range(n_tiles):
        @pl.when(k == n_levels + m)
        def _dot_tile(m=m):
            base = m * tile_s
            tok = jax.lax.broadcasted_iota(jnp.int32, (1, vocab), 1)
            parts = []
            for l in range(n_levels):
                ids = ids_ref[l, base:base + tile_s]
                parts.append((ids[:, None] == tok).astype(jnp.bfloat16))
            onehot = jnp.concatenate(parts, axis=1)
            o_ref[...] = jnp.dot(onehot, cast_ref[...],
                                 preferred_element_type=jnp.float32)


@functools.partial(jax.jit, static_argnames=("vocab",))
def _embed_sum(idx, tbl, *, vocab):
    n_levels, seq = idx.shape
    _, d = tbl.shape
    n_tiles = 2
    tile_s = seq // n_tiles

    body = functools.partial(_two_phase_kernel, n_levels=n_levels,
                             n_tiles=n_tiles, vocab=vocab, tile_s=tile_s)
    return pl.pallas_call(
        body,
        out_shape=jax.ShapeDtypeStruct((seq, d), jnp.float32),
        grid=(n_levels + n_tiles,),
        in_specs=[
            pl.BlockSpec((n_levels, seq), lambda k: (0, 0)),
            pl.BlockSpec((vocab, d),
                         lambda k: (jnp.minimum(k, n_levels - 1), 0)),
        ],
        out_specs=pl.BlockSpec(
            (tile_s, d), lambda k: (jnp.maximum(k - n_levels, 0), 0)),
        scratch_shapes=[
            pltpu.VMEM((n_levels * vocab, d), jnp.bfloat16),
        ],
        compiler_params=pltpu.CompilerParams(
            dimension_semantics=("arbitrary",),
            vmem_limit_bytes=64 * 2**20),
    )(idx, tbl)


def kernel(xi, tables):
    xi = jnp.asarray(xi)
    n_levels = xi.shape[-1] - 1
    idx = jnp.transpose(xi[:, :n_levels]).astype(jnp.int32)
    n_tbl, n_tok, d = tables.shape
    tbl = tables.reshape(n_tbl * n_tok, d)
    return _embed_sum(idx, tbl, vocab=n_tok)
